# Initial kernel scaffold; baseline (speedup 1.0000x reference)
import jax
import jax.numpy as jnp
from jax.experimental import pallas as pl


def kernel(x, edge_index, batch, W_rel1, W_root1, b1, W_rel2, W_root2, b2, W_rel3, W_root3, b3, mlp_W1, mlp_b1, mlp_W2, mlp_b2):
    raise NotImplementedError("write your pallas kernel here")


# speedup vs baseline: 3.5543x; 3.5543x over previous
import functools

import jax
import jax.numpy as jnp
from jax import lax
from jax.experimental import pallas as pl
from jax.experimental.pallas import tpu as pltpu
from jax.experimental.pallas import tpu_sc as plsc

N, E, D, G = 10000, 320000, 128, 16
NC, NS = 2, 16
NW = NC * NS
WIN = 128
EPW = -(-E // (NW * WIN)) * WIN
E_PAD = EPW * NW
N_DUMMY = 16
N_PAD = N + N_DUMMY
ROWS_PER_TILE = 624
ROW_TAIL_OFF = ROWS_PER_TILE * NS
ROW_TAIL = N - ROW_TAIL_OFF


def _sc_scatter_body(h, srcp, dstp, zeros0, out, idx_s, idx_d, rows, acc, sem):
    c = lax.axis_index("c")
    s = lax.axis_index("s")
    wid = c * NS + s
    rs = s * ROWS_PER_TILE

    pltpu.sync_copy(zeros0.at[pl.ds(rs, ROWS_PER_TILE)],
                    acc.at[pl.ds(rs, ROWS_PER_TILE)])

    @pl.when(s == NS - 1)
    def _():
        pltpu.sync_copy(zeros0.at[pl.ds(ROW_TAIL_OFF, ROW_TAIL)],
                        acc.at[pl.ds(ROW_TAIL_OFF, ROW_TAIL)])

    plsc.subcore_barrier()

    base = wid * EPW

    def body(j, carry):
        off = base + j * WIN
        pltpu.sync_copy(srcp.at[pl.ds(off, WIN)], idx_s)
        pltpu.sync_copy(dstp.at[pl.ds(off, WIN)], idx_d)
        pltpu.async_copy(h.at[idx_s], rows, sem).wait()
        pltpu.sync_copy(rows, acc.at[idx_d], add=True)
        return carry

    lax.fori_loop(0, EPW // WIN, body, 0)

    plsc.subcore_barrier()
    pltpu.sync_copy(acc.at[pl.ds(rs, ROWS_PER_TILE)],
                    out.at[c, pl.ds(rs, ROWS_PER_TILE)])

    @pl.when(s == NS - 1)
    def _():
        pltpu.sync_copy(acc.at[pl.ds(ROW_TAIL_OFF, ROW_TAIL)],
                        out.at[c, pl.ds(ROW_TAIL_OFF, ROW_TAIL)])


@functools.cache
def _get_sc_scatter():
    return pl.kernel(
        _sc_scatter_body,
        out_type=jax.ShapeDtypeStruct((NC, N, D), jnp.float32),
        mesh=plsc.VectorSubcoreMesh(core_axis_name="c", subcore_axis_name="s",
                                    num_cores=NC, num_subcores=NS),
        scratch_types=[
            pltpu.VMEM((WIN,), jnp.int32),
            pltpu.VMEM((WIN,), jnp.int32),
            pltpu.VMEM((WIN, D), jnp.float32),
            pltpu.VMEM_SHARED((N_PAD, D), jnp.float32),
            pltpu.SemaphoreType.DMA,
        ],
    )


_NB = 10
_BS = N // _NB
_DN = (((1,), (1,)), ((), ()))


def _conv_body(p_ref, h_ref, wr_ref, wo_ref, b_ref, o_ref):
    agg = p_ref[0] + p_ref[1]
    o_ref[...] = jnp.maximum(
        lax.dot_general(agg, wr_ref[...], _DN,
                        preferred_element_type=jnp.float32)
        + b_ref[...]
        + lax.dot_general(h_ref[...], wo_ref[...], _DN,
                          preferred_element_type=jnp.float32),
        0.0)


def _conv_relu(parts, h, Wr, Wo, b):
    return pl.pallas_call(
        _conv_body,
        grid=(_NB,),
        in_specs=[
            pl.BlockSpec((NC, _BS, D), lambda i: (0, i, 0)),
            pl.BlockSpec((_BS, D), lambda i: (i, 0)),
            pl.BlockSpec((D, D), lambda i: (0, 0)),
            pl.BlockSpec((D, D), lambda i: (0, 0)),
            pl.BlockSpec((1, D), lambda i: (0, 0)),
        ],
        out_specs=pl.BlockSpec((_BS, D), lambda i: (i, 0)),
        out_shape=jax.ShapeDtypeStruct((N, D), jnp.float32),
    )(parts, h, Wr, Wo, b.reshape(1, D))


def _final_body(p_ref, h_ref, wr_ref, wo_ref, b_ref, batch_ref,
                w1_ref, b1_ref, w2_ref, b2_ref, out_ref, pooled_ref):
    i = pl.program_id(0)
    agg = p_ref[0] + p_ref[1]
    h3 = (lax.dot_general(agg, wr_ref[...], _DN,
                          preferred_element_type=jnp.float32)
          + b_ref[...]
          + lax.dot_general(h_ref[...], wo_ref[...], _DN,
                            preferred_element_type=jnp.float32))
    bvec = batch_ref[...]
    neg = jnp.float32(-jnp.inf)
    cols = []
    for g in range(G):
        hg = jnp.where(bvec == g, h3, neg)
        cols.append(jnp.max(hg, axis=0))
    cur = jnp.stack(cols, axis=0)

    @pl.when(i == 0)
    def _():
        pooled_ref[...] = cur

    @pl.when(i > 0)
    def _():
        pooled_ref[...] = jnp.maximum(pooled_ref[...], cur)

    @pl.when(i == _NB - 1)
    def _():
        t1 = lax.dot_general(pooled_ref[...], w1_ref[...], _DN,
                             preferred_element_type=jnp.float32)
        t1 = jnp.maximum(t1 + b1_ref[...], 0.0)
        t2 = lax.dot_general(t1, w2_ref[...], _DN,
                             preferred_element_type=jnp.float32)
        out_ref[...] = t2[:, 0:1] + b2_ref[...]


def _final_stage(parts, h, Wr, Wo, b, batch2d, W1p, b1p, W2p, b2):
    return pl.pallas_call(
        _final_body,
        grid=(_NB,),
        in_specs=[
            pl.BlockSpec((NC, _BS, D), lambda i: (0, i, 0)),
            pl.BlockSpec((_BS, D), lambda i: (i, 0)),
            pl.BlockSpec((D, D), lambda i: (0, 0)),
            pl.BlockSpec((D, D), lambda i: (0, 0)),
            pl.BlockSpec((1, D), lambda i: (0, 0)),
            pl.BlockSpec((_BS, 1), lambda i: (i, 0)),
            pl.BlockSpec((8, D), lambda i: (0, 0)),
            pl.BlockSpec((1, 8), lambda i: (0, 0)),
            pl.BlockSpec((8, 8), lambda i: (0, 0)),
            pl.BlockSpec((1, 1), lambda i: (0, 0)),
        ],
        out_specs=pl.BlockSpec((G, 1), lambda i: (0, 0)),
        out_shape=jax.ShapeDtypeStruct((G, 1), jnp.float32),
        scratch_shapes=[pltpu.VMEM((G, D), jnp.float32)],
    )(parts, h, Wr, Wo, b.reshape(1, D), batch2d, W1p, b1p, W2p,
      b2.reshape(1, 1))


def kernel(x, edge_index, batch,
           W_rel1, W_root1, b1, W_rel2, W_root2, b2, W_rel3, W_root3, b3,
           mlp_W1, mlp_b1, mlp_W2, mlp_b2):
    src = edge_index[0]
    dst = edge_index[1]
    pad = E_PAD - E
    src_p = jnp.concatenate([src, jnp.zeros((pad,), jnp.int32)])
    dst_p = jnp.concatenate(
        [dst, N + (jnp.arange(pad, dtype=jnp.int32) % N_DUMMY)])
    zeros = jnp.zeros((N, D), jnp.float32)
    batch2d = batch.reshape(N, 1)

    W1p = jnp.zeros((8, D), jnp.float32).at[:5].set(mlp_W1)
    b1p = jnp.zeros((1, 8), jnp.float32).at[0, :5].set(mlp_b1)
    W2p = jnp.zeros((8, 8), jnp.float32).at[0, :5].set(mlp_W2[0])

    sc_scatter = _get_sc_scatter()
    parts = sc_scatter(x, src_p, dst_p, zeros)
    h = _conv_relu(parts, x, W_rel1, W_root1, b1)
    parts = sc_scatter(h, src_p, dst_p, zeros)
    h = _conv_relu(parts, h, W_rel2, W_root2, b2)
    parts = sc_scatter(h, src_p, dst_p, zeros)
    return _final_stage(parts, h, W_rel3, W_root3, b3, batch2d,
                        W1p, b1p, W2p, mlp_b2)
